# baseline (speedup 1.0000x reference)
import functools

import jax
import jax.numpy as jnp
from jax import lax
from jax.experimental import pallas as pl
from jax.experimental.pallas import tpu as pltpu
from jax.experimental.pallas import tpu_sc as plsc

_NUM_CORES = 2
_NUM_SUBCORES = 16
_NUM_WORKERS = _NUM_CORES * _NUM_SUBCORES
_NUM_CHUNKS = 4


def kernel(t, pos_embedding):
    (batch,) = t.shape
    vocab, dim = pos_embedding.shape
    b_per_w = batch // _NUM_WORKERS

    mesh = plsc.VectorSubcoreMesh(core_axis_name="c", subcore_axis_name="s")

    @functools.partial(
        pl.kernel,
        mesh=mesh,
        out_type=jax.ShapeDtypeStruct((batch, dim), pos_embedding.dtype),
        scratch_types=[
            pltpu.VMEM_SHARED((vocab, dim), jnp.float32),
            pltpu.VMEM((b_per_w,), jnp.int32),
            pltpu.VMEM((b_per_w, dim), jnp.float32),
            pltpu.SemaphoreType.DMA,
            pltpu.SemaphoreType.DMA,
        ],
        compiler_params=pltpu.CompilerParams(skip_device_barrier=True),
    )
    def gather_kernel(table_hbm, idx_hbm, out_hbm, table_sp, idx_v, rows_v,
                      gsem, wsem):
        sid = lax.axis_index("s")
        wid = sid * _NUM_CORES + lax.axis_index("c")
        base = wid * b_per_w
        chunk = b_per_w // _NUM_CHUNKS

        pltpu.sync_copy(idx_hbm.at[pl.ds(base, b_per_w)], idx_v)
        gathers = [pltpu.async_copy(
            table_hbm.at[idx_v.at[pl.ds(0, chunk)]],
            rows_v.at[pl.ds(0, chunk)],
            gsem,
        )]

        @pl.when(sid < 13)
        def _fill_a():
            pltpu.sync_copy(
                table_hbm.at[pl.ds(sid * 64, 64)],
                table_sp.at[pl.ds(sid * 64, 64)],
            )

        @pl.when(sid >= 13)
        def _fill_b():
            pltpu.sync_copy(
                table_hbm.at[pl.ds(832 + (sid - 13) * 56, 56)],
                table_sp.at[pl.ds(832 + (sid - 13) * 56, 56)],
            )

        plsc.subcore_barrier()
        gathers += [
            pltpu.async_copy(
                table_sp.at[idx_v.at[pl.ds(k * chunk, chunk)]],
                rows_v.at[pl.ds(k * chunk, chunk)],
                gsem,
            )
            for k in range(1, _NUM_CHUNKS)
        ]
        writes = []
        for k in range(_NUM_CHUNKS):
            gathers[k].wait()
            writes.append(pltpu.async_copy(
                rows_v.at[pl.ds(k * chunk, chunk)],
                out_hbm.at[pl.ds(base + k * chunk, chunk)],
                wsem,
            ))
        for w in writes:
            w.wait()

    return gather_kernel(pos_embedding, t.astype(jnp.int32))


# speedup vs baseline: 1.0095x; 1.0095x over previous
import functools

import jax
import jax.numpy as jnp
from jax import lax
from jax.experimental import pallas as pl
from jax.experimental.pallas import tpu as pltpu
from jax.experimental.pallas import tpu_sc as plsc

_NUM_CORES = 2
_NUM_SUBCORES = 16
_NUM_WORKERS = _NUM_CORES * _NUM_SUBCORES
_NUM_CHUNKS = 5
_CHUNK0 = 64


def kernel(t, pos_embedding):
    (batch,) = t.shape
    vocab, dim = pos_embedding.shape
    b_per_w = batch // _NUM_WORKERS

    mesh = plsc.VectorSubcoreMesh(core_axis_name="c", subcore_axis_name="s")

    @functools.partial(
        pl.kernel,
        mesh=mesh,
        out_type=jax.ShapeDtypeStruct((batch, dim), pos_embedding.dtype),
        scratch_types=[
            pltpu.VMEM_SHARED((vocab, dim), jnp.float32),
            pltpu.VMEM((b_per_w,), jnp.int32),
            pltpu.VMEM((b_per_w, dim), jnp.float32),
            pltpu.SemaphoreType.DMA,
            pltpu.SemaphoreType.DMA,
        ],
    )
    def gather_kernel(table_hbm, idx_hbm, out_hbm, table_sp, idx_v, rows_v,
                      gsem, wsem):
        sid = lax.axis_index("s")
        wid = sid * _NUM_CORES + lax.axis_index("c")
        base = wid * b_per_w
        sizes = [_CHUNK0] + [(b_per_w - _CHUNK0) // (_NUM_CHUNKS - 1)] * (
            _NUM_CHUNKS - 1)
        offs = [sum(sizes[:k]) for k in range(_NUM_CHUNKS)]

        pltpu.sync_copy(idx_hbm.at[pl.ds(base, b_per_w)], idx_v)
        gathers = [pltpu.async_copy(
            table_hbm.at[idx_v.at[pl.ds(0, _CHUNK0)]],
            rows_v.at[pl.ds(0, _CHUNK0)],
            gsem,
        )]

        @pl.when(sid < 13)
        def _fill_a():
            pltpu.sync_copy(
                table_hbm.at[pl.ds(sid * 64, 64)],
                table_sp.at[pl.ds(sid * 64, 64)],
            )

        @pl.when(sid >= 13)
        def _fill_b():
            pltpu.sync_copy(
                table_hbm.at[pl.ds(832 + (sid - 13) * 56, 56)],
                table_sp.at[pl.ds(832 + (sid - 13) * 56, 56)],
            )

        plsc.subcore_barrier()
        gathers += [
            pltpu.async_copy(
                table_sp.at[idx_v.at[pl.ds(offs[k], sizes[k])]],
                rows_v.at[pl.ds(offs[k], sizes[k])],
                gsem,
            )
            for k in range(1, _NUM_CHUNKS)
        ]
        writes = []
        for k in range(_NUM_CHUNKS):
            gathers[k].wait()
            writes.append(pltpu.async_copy(
                rows_v.at[pl.ds(offs[k], sizes[k])],
                out_hbm.at[pl.ds(base + offs[k], sizes[k])],
                wsem,
            ))
        for w in writes:
            w.wait()

    return gather_kernel(pos_embedding, t.astype(jnp.int32))


# speedup vs baseline: 1.0120x; 1.0025x over previous
import functools

import jax
import jax.numpy as jnp
from jax import lax
from jax.experimental import pallas as pl
from jax.experimental.pallas import tpu as pltpu
from jax.experimental.pallas import tpu_sc as plsc

_NUM_CORES = 2
_NUM_SUBCORES = 16
_NUM_WORKERS = _NUM_CORES * _NUM_SUBCORES
_NUM_CHUNKS = 5
_CHUNK0 = 64


def kernel(t, pos_embedding):
    (batch,) = t.shape
    vocab, dim = pos_embedding.shape
    assert vocab == 1000 and batch % _NUM_WORKERS == 0
    b_per_w = batch // _NUM_WORKERS

    mesh = plsc.VectorSubcoreMesh(core_axis_name="c", subcore_axis_name="s")

    @functools.partial(
        pl.kernel,
        mesh=mesh,
        out_type=jax.ShapeDtypeStruct((batch, dim), pos_embedding.dtype),
        scratch_types=[
            pltpu.VMEM_SHARED((vocab, dim), jnp.float32),
            pltpu.VMEM((b_per_w,), jnp.int32),
            pltpu.VMEM((b_per_w, dim), jnp.float32),
            pltpu.SemaphoreType.DMA,
            pltpu.SemaphoreType.DMA,
        ],
    )
    def gather_kernel(table_hbm, idx_hbm, out_hbm, table_sp, idx_v, rows_v,
                      gsem, wsem):
        sid = lax.axis_index("s")
        wid = sid * _NUM_CORES + lax.axis_index("c")
        base = wid * b_per_w
        sizes = [_CHUNK0] + [(b_per_w - _CHUNK0) // (_NUM_CHUNKS - 1)] * (
            _NUM_CHUNKS - 1)
        offs = [sum(sizes[:k]) for k in range(_NUM_CHUNKS)]

        pltpu.sync_copy(idx_hbm.at[pl.ds(base, b_per_w)], idx_v)
        gathers = [pltpu.async_copy(
            table_hbm.at[idx_v.at[pl.ds(0, _CHUNK0)]],
            rows_v.at[pl.ds(0, _CHUNK0)],
            gsem,
        )]

        @pl.when(sid < 13)
        def _fill_a():
            pltpu.sync_copy(
                table_hbm.at[pl.ds(sid * 64, 64)],
                table_sp.at[pl.ds(sid * 64, 64)],
            )

        @pl.when(sid >= 13)
        def _fill_b():
            pltpu.sync_copy(
                table_hbm.at[pl.ds(832 + (sid - 13) * 56, 56)],
                table_sp.at[pl.ds(832 + (sid - 13) * 56, 56)],
            )

        plsc.subcore_barrier()
        gathers += [
            pltpu.async_copy(
                table_sp.at[idx_v.at[pl.ds(offs[k], sizes[k])]],
                rows_v.at[pl.ds(offs[k], sizes[k])],
                gsem,
            )
            for k in range(1, _NUM_CHUNKS)
        ]
        writes = []
        for k in range(_NUM_CHUNKS):
            gathers[k].wait()
            writes.append(pltpu.async_copy(
                rows_v.at[pl.ds(offs[k], sizes[k])],
                out_hbm.at[pl.ds(base + offs[k], sizes[k])],
                wsem,
            ))
        for w in writes:
            w.wait()

    return gather_kernel(pos_embedding, t.astype(jnp.int32))
